# baseline (speedup 1.0000x reference)
import functools

import jax
import jax.numpy as jnp
from jax import lax
from jax.experimental import pallas as pl
from jax.experimental.pallas import tpu as pltpu
from jax.experimental.pallas import tpu_sc as plsc

_WIN = 512
_WBITS = 9
_RING = 2


@functools.lru_cache(maxsize=None)
def _build(B, V, D):
    info = plsc.get_sparse_core_info()
    NC, NS = info.num_cores, info.num_subcores
    NW = NC * NS
    n_win = (V + _WIN - 1) // _WIN
    n_full = V // _WIN
    tail = V - n_full * _WIN
    max_loc_win = (((n_win + NW - 1) // NW + 1) + 15) & ~15
    cap = B + 16 * max_loc_win
    cap = (cap + 127) & ~127

    mesh = plsc.VectorSubcoreMesh(core_axis_name="c", subcore_axis_name="s")
    i32 = jnp.int32

    @functools.partial(
        pl.kernel,
        mesh=mesh,
        out_type=jax.ShapeDtypeStruct((B + 16, 128), jnp.float32),
        compiler_params=pltpu.CompilerParams(needs_layout_passes=False),
        scratch_types=[
            pltpu.VMEM((B,), i32),
            pltpu.VMEM((cap,), i32),
            pltpu.VMEM((max_loc_win,), i32),
            pltpu.VMEM((max_loc_win,), i32),
            pltpu.VMEM((max_loc_win,), i32),
            pltpu.VMEM((_RING, D, _WIN), jnp.float32),
            pltpu.VMEM((2, 16, 128), jnp.float32),
            pltpu.SemaphoreType.DMA,
            pltpu.SemaphoreType.DMA,
        ],
    )
    def emb_kernel(
        tT_hbm, x_hbm, tail_hbm, out_hbm, idx_v, sorted_v, hist, offs, run,
        win_v, rowbuf, sem, sem2
    ):
        wid = lax.axis_index("s") * NC + lax.axis_index("c")
        lo = (n_win * wid) // NW
        hi = (n_win * (wid + 1)) // NW
        hi_c = jnp.minimum(hi, n_full)
        nb = max_loc_win

        def fire(w):
            pltpu.async_copy(
                tT_hbm.at[:, pl.ds(w * _WIN, _WIN)], win_v.at[w & (_RING - 1)],
                sem
            )

        for j in range(_RING - 1):
            @pl.when(lo + j < hi_c)
            def _(j=j):
                fire(lo + j)

        pltpu.sync_copy(x_hbm, idx_v)

        zero16 = jnp.zeros((16,), i32)
        iota16 = lax.iota(i32, 16)
        ones16 = jnp.ones((16,), i32)

        for v in range(nb // 16):
            hist[pl.ds(v * 16, 16)] = zero16
            run[pl.ds(v * 16, 16)] = zero16

        def wloc(u16):
            w16 = jax.lax.shift_right_logical(u16, _WBITS)
            m = (w16 >= lo) & (w16 < hi)
            wl = jnp.clip(w16 - lo, 0, nb - 1)
            return wl, m

        def hist_body(i, _):
            u16 = idx_v[pl.ds(i * 16, 16)]
            wl, m = wloc(u16)
            rank, last = plsc.scan_count(wl, m)
            plsc.addupdate_scatter(hist, [wl], rank, mask=m & last)
            return 0

        lax.fori_loop(0, B // 16, hist_body, 0)

        carry = jnp.zeros((), i32)
        for v in range(nb // 16):
            h = hist[pl.ds(v * 16, 16)]
            hp = (h + 15) & ~15
            inc = plsc.cumsum(hp)
            excl = inc - hp + carry
            offs[pl.ds(v * 16, 16)] = excl
            run[pl.ds(v * 16, 16)] = excl
            carry = carry + inc[15]

        def scat_body(i, _):
            u16 = idx_v[pl.ds(i * 16, 16)]
            wl, m = wloc(u16)
            rank, last = plsc.scan_count(wl, m)
            base = plsc.load_gather(run, [wl], mask=m)
            pos = jnp.clip(base + rank - 1, 0, cap - 1)
            b16 = i * 16 + iota16
            loc16 = u16 & (_WIN - 1)
            packed = jax.lax.shift_left(b16, _WBITS) | loc16
            plsc.store_scatter(sorted_v, [pos], packed, mask=m)
            plsc.addupdate_scatter(run, [wl], rank, mask=m & last)
            return 0

        lax.fori_loop(0, B // 16, scat_body, 0)

        dummy16 = jnp.full((16,), B, i32)

        def wait_one_scatter():
            pltpu.make_async_copy(
                rowbuf.at[0], out_hbm.at[dummy16], sem2
            ).wait()

        def process_window(w, buf, cnt, force_n=None):
            wl = w - lo
            n = plsc.load_gather(hist, [jnp.full((16,), wl, i32)])[0]
            n = jnp.clip(n, 0, B)
            if force_n is not None:
                n = force_n(n)
            off0 = plsc.load_gather(offs, [jnp.full((16,), wl, i32)])[0]
            off0 = jnp.clip(off0, 0, cap - 16)

            def chunk_body(mc, cnt):
                p = cnt & 1

                @pl.when(cnt >= 2)
                def _():
                    wait_one_scatter()

                chunk = sorted_v[pl.ds(off0 + mc * 16, 16)]
                loc16 = chunk & (_WIN - 1)
                b16 = jax.lax.shift_right_logical(chunk, _WBITS)
                valid = iota16 < (n - mc * 16)
                bidx = jnp.where(valid, jnp.clip(b16, 0, B - 1), B + wid % 16)
                for k in range(16):
                    lk = loc16[k]
                    for g in range(D // 16):
                        c16 = iota16 + g * 16
                        vals = plsc.load_gather(
                            win_v,
                            [jnp.full((16,), buf, i32), c16,
                             jnp.full((16,), lk, i32)],
                        )
                        sig = 1.0 / (1.0 + jnp.exp(-vals))
                        plsc.store_scatter(
                            rowbuf,
                            [jnp.full((16,), p, i32), jnp.full((16,), k, i32),
                             c16],
                            sig,
                        )
                pltpu.async_copy(rowbuf.at[p], out_hbm.at[bidx], sem2)
                return cnt + 1

            return lax.fori_loop(0, (n + 15) >> 4, chunk_body, cnt)

        def win_body(w, cnt):
            pltpu.make_async_copy(
                tT_hbm.at[:, pl.ds(0, _WIN)], win_v.at[w & (_RING - 1)], sem
            ).wait()

            @pl.when(w + _RING - 1 < hi_c)
            def _():
                fire(w + _RING - 1)

            return process_window(w, w & (_RING - 1), cnt)

        cnt = lax.fori_loop(lo, hi_c, win_body, jnp.zeros((), i32))

        pltpu.sync_copy(tail_hbm, win_v.at[0])
        cnt = process_window(
            jnp.full((), n_full, i32),
            jnp.zeros((), i32),
            cnt,
            force_n=lambda n: jnp.where(hi == n_win, n, 0),
        )

        @pl.when(cnt >= 1)
        def _():
            wait_one_scatter()

        @pl.when(cnt >= 2)
        def _():
            wait_one_scatter()

    return emb_kernel


def kernel(x, table):
    B = x.shape[0]
    V, D = table.shape
    emb_kernel = _build(B, V, D)
    n_full = V // _WIN
    tailT = jnp.pad(
        table.T[:, n_full * _WIN :], ((0, 0), (0, _WIN - (V - n_full * _WIN)))
    )
    out128 = emb_kernel(table.T, x.astype(jnp.int32), tailT)
    return out128[:B, :D]


# speedup vs baseline: 1.0020x; 1.0020x over previous
import functools

import jax
import jax.numpy as jnp
from jax import lax
from jax.experimental import pallas as pl
from jax.experimental.pallas import tpu as pltpu
from jax.experimental.pallas import tpu_sc as plsc

_WIN = 512
_WBITS = 9
_RING = 2


@functools.lru_cache(maxsize=None)
def _build(B, V, D):
    info = plsc.get_sparse_core_info()
    NC, NS = info.num_cores, info.num_subcores
    NW = NC * NS
    n_win = (V + _WIN - 1) // _WIN
    n_full = V // _WIN
    tail = V - n_full * _WIN
    max_loc_win = (((n_win + NW - 1) // NW + 1) + 15) & ~15
    cap = B + 16 * max_loc_win
    cap = (cap + 127) & ~127

    mesh = plsc.VectorSubcoreMesh(core_axis_name="c", subcore_axis_name="s")
    i32 = jnp.int32

    @functools.partial(
        pl.kernel,
        mesh=mesh,
        out_type=jax.ShapeDtypeStruct((B + 16, 128), jnp.float32),
        compiler_params=pltpu.CompilerParams(needs_layout_passes=False),
        scratch_types=[
            pltpu.VMEM((B,), i32),
            pltpu.VMEM((cap,), i32),
            pltpu.VMEM((max_loc_win,), i32),
            pltpu.VMEM((max_loc_win,), i32),
            pltpu.VMEM((max_loc_win,), i32),
            pltpu.VMEM((_RING, D, _WIN), jnp.float32),
            pltpu.VMEM((2, 16, 128), jnp.float32),
            pltpu.SemaphoreType.DMA,
            pltpu.SemaphoreType.DMA,
        ],
    )
    def emb_kernel(
        tT_hbm, x_hbm, tail_hbm, out_hbm, idx_v, sorted_v, hist, offs, run,
        win_v, rowbuf, sem, sem2
    ):
        wid = lax.axis_index("s") * NC + lax.axis_index("c")
        lo = (n_win * wid) // NW
        hi = (n_win * (wid + 1)) // NW
        hi_c = jnp.minimum(hi, n_full)
        nb = max_loc_win

        def fire(w):
            pltpu.async_copy(
                tT_hbm.at[:, pl.ds(w * _WIN, _WIN)], win_v.at[w & (_RING - 1)],
                sem
            )

        for j in range(_RING - 1):
            @pl.when(lo + j < hi_c)
            def _(j=j):
                fire(lo + j)

        pltpu.sync_copy(x_hbm, idx_v)

        zero16 = jnp.zeros((16,), i32)
        iota16 = lax.iota(i32, 16)
        ones16 = jnp.ones((16,), i32)

        for v in range(nb // 16):
            hist[pl.ds(v * 16, 16)] = zero16
            run[pl.ds(v * 16, 16)] = zero16

        def wloc(u16):
            w16 = jax.lax.shift_right_logical(u16, _WBITS)
            m = (w16 >= lo) & (w16 < hi)
            wl = jnp.clip(w16 - lo, 0, nb - 1)
            return wl, m

        def hist_body(i, _):
            u16 = idx_v[pl.ds(i * 16, 16)]
            wl, m = wloc(u16)
            rank, last = plsc.scan_count(wl, m)
            plsc.addupdate_scatter(hist, [wl], rank, mask=m & last)
            return 0

        lax.fori_loop(0, B // 16, hist_body, 0, unroll=4)

        carry = jnp.zeros((), i32)
        for v in range(nb // 16):
            h = hist[pl.ds(v * 16, 16)]
            hp = (h + 15) & ~15
            inc = plsc.cumsum(hp)
            excl = inc - hp + carry
            offs[pl.ds(v * 16, 16)] = excl
            run[pl.ds(v * 16, 16)] = excl
            carry = carry + inc[15]

        def scat_body(i, _):
            u16 = idx_v[pl.ds(i * 16, 16)]
            wl, m = wloc(u16)
            rank, last = plsc.scan_count(wl, m)
            base = plsc.load_gather(run, [wl], mask=m)
            pos = jnp.clip(base + rank - 1, 0, cap - 1)
            b16 = i * 16 + iota16
            loc16 = u16 & (_WIN - 1)
            packed = jax.lax.shift_left(b16, _WBITS) | loc16
            plsc.store_scatter(sorted_v, [pos], packed, mask=m)
            plsc.addupdate_scatter(run, [wl], rank, mask=m & last)
            return 0

        lax.fori_loop(0, B // 16, scat_body, 0, unroll=4)

        dummy16 = jnp.full((16,), B, i32)

        def wait_one_scatter():
            pltpu.make_async_copy(
                rowbuf.at[0], out_hbm.at[dummy16], sem2
            ).wait()

        def process_window(w, buf, cnt, force_n=None):
            wl = w - lo
            n = plsc.load_gather(hist, [jnp.full((16,), wl, i32)])[0]
            n = jnp.clip(n, 0, B)
            if force_n is not None:
                n = force_n(n)
            off0 = plsc.load_gather(offs, [jnp.full((16,), wl, i32)])[0]
            off0 = jnp.clip(off0, 0, cap - 16)

            def chunk_body(mc, cnt):
                p = cnt & 1

                @pl.when(cnt >= 2)
                def _():
                    wait_one_scatter()

                chunk = sorted_v[pl.ds(off0 + mc * 16, 16)]
                loc16 = chunk & (_WIN - 1)
                b16 = jax.lax.shift_right_logical(chunk, _WBITS)
                valid = iota16 < (n - mc * 16)
                bidx = jnp.where(valid, jnp.clip(b16, 0, B - 1), B + wid % 16)
                for k in range(16):
                    lk = loc16[k]
                    for g in range(D // 16):
                        c16 = iota16 + g * 16
                        vals = plsc.load_gather(
                            win_v,
                            [jnp.full((16,), buf, i32), c16,
                             jnp.full((16,), lk, i32)],
                        )
                        sig = 1.0 / (1.0 + jnp.exp(-vals))
                        plsc.store_scatter(
                            rowbuf,
                            [jnp.full((16,), p, i32), jnp.full((16,), k, i32),
                             c16],
                            sig,
                        )
                pltpu.async_copy(rowbuf.at[p], out_hbm.at[bidx], sem2)
                return cnt + 1

            return lax.fori_loop(0, (n + 15) >> 4, chunk_body, cnt)

        def win_body(w, cnt):
            pltpu.make_async_copy(
                tT_hbm.at[:, pl.ds(0, _WIN)], win_v.at[w & (_RING - 1)], sem
            ).wait()

            @pl.when(w + _RING - 1 < hi_c)
            def _():
                fire(w + _RING - 1)

            return process_window(w, w & (_RING - 1), cnt)

        cnt = lax.fori_loop(lo, hi_c, win_body, jnp.zeros((), i32))

        pltpu.sync_copy(tail_hbm, win_v.at[0])
        cnt = process_window(
            jnp.full((), n_full, i32),
            jnp.zeros((), i32),
            cnt,
            force_n=lambda n: jnp.where(hi == n_win, n, 0),
        )

        @pl.when(cnt >= 1)
        def _():
            wait_one_scatter()

        @pl.when(cnt >= 2)
        def _():
            wait_one_scatter()

    return emb_kernel


def kernel(x, table):
    B = x.shape[0]
    V, D = table.shape
    emb_kernel = _build(B, V, D)
    n_full = V // _WIN
    tailT = jnp.pad(
        table.T[:, n_full * _WIN :], ((0, 0), (0, _WIN - (V - n_full * _WIN)))
    )
    out128 = emb_kernel(table.T, x.astype(jnp.int32), tailT)
    return out128[:B, :D]


# speedup vs baseline: 1.0137x; 1.0116x over previous
import functools

import jax
import jax.numpy as jnp
from jax import lax
from jax.experimental import pallas as pl
from jax.experimental.pallas import tpu as pltpu
from jax.experimental.pallas import tpu_sc as plsc

_WIN = 512
_WBITS = 9
_RING = 2


@functools.lru_cache(maxsize=None)
def _build(B, V, D):
    info = plsc.get_sparse_core_info()
    NC, NS = info.num_cores, info.num_subcores
    NW = NC * NS
    n_win = (V + _WIN - 1) // _WIN
    n_full = V // _WIN
    tail = V - n_full * _WIN
    max_loc_win = (((n_win + NW - 1) // NW + 1) + 15) & ~15
    cap = B + 16 * max_loc_win
    cap = (cap + 127) & ~127

    mesh = plsc.VectorSubcoreMesh(core_axis_name="c", subcore_axis_name="s")
    i32 = jnp.int32

    @functools.partial(
        pl.kernel,
        mesh=mesh,
        out_type=jax.ShapeDtypeStruct((B + 16, 128), jnp.float32),
        compiler_params=pltpu.CompilerParams(needs_layout_passes=False),
        scratch_types=[
            pltpu.VMEM((B,), i32),
            pltpu.VMEM((cap,), i32),
            pltpu.VMEM((max_loc_win,), i32),
            pltpu.VMEM((max_loc_win,), i32),
            pltpu.VMEM((max_loc_win,), i32),
            pltpu.VMEM((_RING, D, _WIN), jnp.float32),
            pltpu.VMEM((2, 16, 128), jnp.float32),
            pltpu.SemaphoreType.DMA,
            pltpu.SemaphoreType.DMA,
        ],
    )
    def emb_kernel(
        tT_hbm, x_hbm, tail_hbm, out_hbm, idx_v, sorted_v, hist, offs, run,
        win_v, rowbuf, sem, sem2
    ):
        wid = lax.axis_index("s") * NC + lax.axis_index("c")
        lo = (n_win * wid) // NW
        hi = (n_win * (wid + 1)) // NW
        hi_c = jnp.minimum(hi, n_full)
        nb = max_loc_win

        def fire(w):
            pltpu.async_copy(
                tT_hbm.at[:, pl.ds(w * _WIN, _WIN)], win_v.at[w & (_RING - 1)],
                sem
            )

        for j in range(_RING - 1):
            @pl.when(lo + j < hi_c)
            def _(j=j):
                fire(lo + j)

        pltpu.sync_copy(x_hbm, idx_v)

        zero16 = jnp.zeros((16,), i32)
        iota16 = lax.iota(i32, 16)
        ones16 = jnp.ones((16,), i32)

        for v in range(nb // 16):
            hist[pl.ds(v * 16, 16)] = zero16
            run[pl.ds(v * 16, 16)] = zero16

        def wloc(u16):
            w16 = jax.lax.shift_right_logical(u16, _WBITS)
            m = (w16 >= lo) & (w16 < hi)
            wl = jnp.clip(w16 - lo, 0, nb - 1)
            return wl, m

        def hist_body(i, _):
            u16 = idx_v[pl.ds(i * 16, 16)]
            wl, m = wloc(u16)
            rank, last = plsc.scan_count(wl, m)
            plsc.addupdate_scatter(hist, [wl], rank, mask=m & last)
            return 0

        lax.fori_loop(0, B // 16, hist_body, 0, unroll=4)

        carry = jnp.zeros((), i32)
        for v in range(nb // 16):
            h = hist[pl.ds(v * 16, 16)]
            hp = (h + 15) & ~15
            inc = plsc.cumsum(hp)
            excl = inc - hp + carry
            offs[pl.ds(v * 16, 16)] = excl
            run[pl.ds(v * 16, 16)] = excl
            carry = carry + inc[15]

        def scat_body(i, _):
            u16 = idx_v[pl.ds(i * 16, 16)]
            wl, m = wloc(u16)
            rank, last = plsc.scan_count(wl, m)
            base = plsc.load_gather(run, [wl], mask=m)
            pos = jnp.clip(base + rank - 1, 0, cap - 1)
            b16 = i * 16 + iota16
            loc16 = u16 & (_WIN - 1)
            packed = jax.lax.shift_left(b16, _WBITS) | loc16
            plsc.store_scatter(sorted_v, [pos], packed, mask=m)
            plsc.addupdate_scatter(run, [wl], rank, mask=m & last)
            return 0

        lax.fori_loop(0, B // 16, scat_body, 0, unroll=4)

        dummy16 = jnp.full((16,), B, i32)

        def wait_one_scatter():
            pltpu.make_async_copy(
                rowbuf.at[0], out_hbm.at[dummy16], sem2
            ).wait()

        def process_window(w, buf, cnt, force_n=None):
            wl = w - lo
            n = plsc.load_gather(hist, [jnp.full((16,), wl, i32)])[0]
            n = jnp.clip(n, 0, B)
            if force_n is not None:
                n = force_n(n)
            off0 = plsc.load_gather(offs, [jnp.full((16,), wl, i32)])[0]
            off0 = jnp.clip(off0, 0, cap - 16)

            def chunk_body(mc, cnt):
                p = cnt & 1

                @pl.when(cnt >= 2)
                def _():
                    wait_one_scatter()

                chunk = sorted_v[pl.ds(off0 + mc * 16, 16)]
                loc16 = chunk & (_WIN - 1)
                b16 = jax.lax.shift_right_logical(chunk, _WBITS)
                valid = iota16 < (n - mc * 16)
                bidx = jnp.where(valid, jnp.clip(b16, 0, B - 1), B + wid % 16)
                for k0 in (0, 8):
                    sigs = []
                    for k in range(k0, k0 + 8):
                        lk = loc16[k]
                        for g in range(D // 16):
                            c16 = iota16 + g * 16
                            vals = plsc.load_gather(
                                win_v,
                                [jnp.full((16,), buf, i32), c16,
                                 jnp.full((16,), lk, i32)],
                            )
                            sigs.append(1.0 / (1.0 + jnp.exp(-vals)))
                    for k in range(k0, k0 + 8):
                        for g in range(D // 16):
                            c16 = iota16 + g * 16
                            plsc.store_scatter(
                                rowbuf,
                                [jnp.full((16,), p, i32),
                                 jnp.full((16,), k, i32), c16],
                                sigs[(k - k0) * (D // 16) + g],
                            )
                pltpu.async_copy(rowbuf.at[p], out_hbm.at[bidx], sem2)
                return cnt + 1

            return lax.fori_loop(0, (n + 15) >> 4, chunk_body, cnt)

        def win_body(w, cnt):
            pltpu.make_async_copy(
                tT_hbm.at[:, pl.ds(0, _WIN)], win_v.at[w & (_RING - 1)], sem
            ).wait()

            @pl.when(w + _RING - 1 < hi_c)
            def _():
                fire(w + _RING - 1)

            return process_window(w, w & (_RING - 1), cnt)

        cnt = lax.fori_loop(lo, hi_c, win_body, jnp.zeros((), i32))

        pltpu.sync_copy(tail_hbm, win_v.at[0])
        cnt = process_window(
            jnp.full((), n_full, i32),
            jnp.zeros((), i32),
            cnt,
            force_n=lambda n: jnp.where(hi == n_win, n, 0),
        )

        @pl.when(cnt >= 1)
        def _():
            wait_one_scatter()

        @pl.when(cnt >= 2)
        def _():
            wait_one_scatter()

    return emb_kernel


def kernel(x, table):
    B = x.shape[0]
    V, D = table.shape
    emb_kernel = _build(B, V, D)
    n_full = V // _WIN
    tailT = jnp.pad(
        table.T[:, n_full * _WIN :], ((0, 0), (0, _WIN - (V - n_full * _WIN)))
    )
    out128 = emb_kernel(table.T, x.astype(jnp.int32), tailT)
    return out128[:B, :D]
